# baseline (speedup 1.0000x reference)
import jax
import jax.numpy as jnp
from jax.experimental import pallas as pl
from jax.experimental.pallas import tpu as pltpu

_MAX_DET = 300
_SCORE_THR = 0.05
_IOU_THR = 0.5
_ROWS = 160
_LANES = 128
_NPAD = _ROWS * _LANES
_NBLK = _ROWS // 8
_NEG = float("-inf")
_BIG = 2**30


def _nms_kernel(cls_ref, bx_ref, ob_ref, os_ref, ol_ref,
                cur_ref, lab_ref, bmax_ref, kx1_ref, ky1_ref, kx2_ref,
                ky2_ref, ka_ref):
    C = cls_ref.shape[0]
    shp = (_ROWS, _LANES)

    def cls_body(c, carry):
        mx, am = carry
        x = cls_ref[c]
        upd = x > mx
        mx = jnp.where(upd, x, mx)
        am = jnp.where(upd, jnp.full(shp, c, jnp.int32), am)
        return mx, am

    scores, labels = jax.lax.fori_loop(
        0, C, cls_body,
        (jnp.full(shp, _NEG, jnp.float32), jnp.zeros(shp, jnp.int32)))

    cur = jnp.where(scores > _SCORE_THR, scores, _NEG)
    cur_ref[...] = cur
    lab_ref[...] = labels

    sub8 = jax.lax.broadcasted_iota(jnp.int32, (8, _LANES), 0)
    lane8 = jax.lax.broadcasted_iota(jnp.int32, (8, _LANES), 1)
    biota = sub8 * _LANES + lane8
    lane1 = jax.lax.broadcasted_iota(jnp.int32, (1, _LANES), 1)

    brow = jnp.full((1, _LANES), _NEG, jnp.float32)
    for b in range(_NBLK):
        mb = jnp.max(cur[b * 8:(b + 1) * 8, :])
        brow = jnp.where(lane1 == b, mb, brow)
    bmax_ref[...] = jnp.where(sub8 == 0, jnp.broadcast_to(brow, (8, _LANES)),
                              jnp.full((8, _LANES), _NEG, jnp.float32))

    ob_ref[...] = jnp.full((_MAX_DET, 4), -1.0, jnp.float32)
    os_ref[...] = jnp.full((_MAX_DET, 1), -1.0, jnp.float32)
    ol_ref[...] = jnp.full((_MAX_DET, 1), -1, jnp.int32)

    kiota = biota

    def cond(carry):
        return carry[1]

    def body(carry):
        t, _ = carry
        bm = bmax_ref[...]
        m = jnp.max(bm)
        alive = m > _NEG
        bb = jnp.min(jnp.where(bm == m, biota, _BIG))
        bb = jnp.where(alive, bb, 0)
        blk = cur_ref[pl.ds(bb * 8, 8), :]
        libn = jnp.min(jnp.where(blk == m, biota, _BIG))
        libn = jnp.where(alive, libn, 0)
        c = libn % _LANES
        r = bb * 8 + libn // _LANES
        newblk = jnp.where(biota == libn, _NEG, blk)
        cur_ref[pl.ds(bb * 8, 8), :] = newblk
        nbm = jnp.max(newblk)
        bmax_ref[...] = jnp.where((sub8 == 0) & (lane8 == bb), nbm, bm)
        selc = lane1 == c
        bxr = bx_ref[:, pl.ds(r, 1), :]
        x1i = jnp.sum(jnp.where(selc, bxr[0], 0.0))
        y1i = jnp.sum(jnp.where(selc, bxr[1], 0.0))
        x2i = jnp.sum(jnp.where(selc, bxr[2], 0.0))
        y2i = jnp.sum(jnp.where(selc, bxr[3], 0.0))
        ai = (x2i - x1i) * (y2i - y1i)
        kx1 = kx1_ref[...]
        ky1 = ky1_ref[...]
        kx2 = kx2_ref[...]
        ky2 = ky2_ref[...]
        ka = ka_ref[...]
        xx1 = jnp.maximum(kx1, x1i)
        yy1 = jnp.maximum(ky1, y1i)
        xx2 = jnp.minimum(kx2, x2i)
        yy2 = jnp.minimum(ky2, y2i)
        inter = jnp.maximum(0.0, xx2 - xx1) * jnp.maximum(0.0, yy2 - yy1)
        iou = inter / (ka + ai - inter + 1e-8)
        supp = jnp.any((iou > _IOU_THR) & (kiota < t))
        accept = alive & jnp.logical_not(supp)

        @pl.when(accept)
        def _store():
            row = jnp.concatenate(
                [jnp.full((1, 1), v, jnp.float32)
                 for v in (x1i, y1i, x2i, y2i)], axis=1)
            ob_ref[pl.ds(t, 1), :] = row
            os_ref[pl.ds(t, 1), :] = jnp.full((1, 1), m, jnp.float32)
            lrow = lab_ref[pl.ds(r, 1), :]
            li = jnp.sum(jnp.where(selc, lrow, 0))
            ol_ref[pl.ds(t, 1), :] = jnp.full((1, 1), li, jnp.int32)
            kx1_ref[...] = jnp.where(kiota == t, x1i, kx1)
            ky1_ref[...] = jnp.where(kiota == t, y1i, ky1)
            kx2_ref[...] = jnp.where(kiota == t, x2i, kx2)
            ky2_ref[...] = jnp.where(kiota == t, y2i, ky2)
            ka_ref[...] = jnp.where(kiota == t, ai, ka)

        t2 = t + accept.astype(jnp.int32)
        return t2, alive & (t2 < _MAX_DET)

    jax.lax.while_loop(cond, body, (jnp.int32(0), jnp.bool_(True)))


@jax.jit
def kernel(boxes, classification):
    B, N, C = classification.shape
    pad = _NPAD - N
    cls_t = jnp.pad(classification, ((0, 0), (0, pad), (0, 0)),
                    constant_values=-1.0)
    cls_t = cls_t.transpose(0, 2, 1).reshape(B, C, _ROWS, _LANES)
    bx = jnp.pad(boxes, ((0, 0), (0, pad), (0, 0)))
    bx = bx.transpose(0, 2, 1).reshape(B, 4, _ROWS, _LANES)

    ob, os_, ol = pl.pallas_call(
        _nms_kernel,
        grid=(B,),
        in_specs=[
            pl.BlockSpec((None, C, _ROWS, _LANES), lambda b: (b, 0, 0, 0)),
            pl.BlockSpec((None, 4, _ROWS, _LANES), lambda b: (b, 0, 0, 0)),
        ],
        out_specs=[
            pl.BlockSpec((None, _MAX_DET, 4), lambda b: (b, 0, 0)),
            pl.BlockSpec((None, _MAX_DET, 1), lambda b: (b, 0, 0)),
            pl.BlockSpec((None, _MAX_DET, 1), lambda b: (b, 0, 0)),
        ],
        out_shape=[
            jax.ShapeDtypeStruct((B, _MAX_DET, 4), jnp.float32),
            jax.ShapeDtypeStruct((B, _MAX_DET, 1), jnp.float32),
            jax.ShapeDtypeStruct((B, _MAX_DET, 1), jnp.int32),
        ],
        scratch_shapes=[
            pltpu.VMEM((_ROWS, _LANES), jnp.float32),
            pltpu.VMEM((_ROWS, _LANES), jnp.int32),
            pltpu.VMEM((8, _LANES), jnp.float32),
            pltpu.VMEM((8, _LANES), jnp.float32),
            pltpu.VMEM((8, _LANES), jnp.float32),
            pltpu.VMEM((8, _LANES), jnp.float32),
            pltpu.VMEM((8, _LANES), jnp.float32),
            pltpu.VMEM((8, _LANES), jnp.float32),
        ],
    )(cls_t, bx)
    return ob, os_.reshape(B, _MAX_DET), ol.reshape(B, _MAX_DET)


# speedup vs baseline: 1.4264x; 1.4264x over previous
import jax
import jax.numpy as jnp
from jax.experimental import pallas as pl
from jax.experimental.pallas import tpu as pltpu

_MAX_DET = 300
_SCORE_THR = 0.05
_IOU_THR = 0.5
_ROWS = 160
_LANES = 128
_NPAD = _ROWS * _LANES
_NEG = float("-inf")
_BIG = 2**30


def _nms_kernel(cls_ref, bx_ref, ob_ref, os_ref, ol_ref, lab_ref):
    C = cls_ref.shape[0]
    shp = (_ROWS, _LANES)

    def cls_body(c, carry):
        mx, am = carry
        x = cls_ref[c]
        upd = x > mx
        mx = jnp.where(upd, x, mx)
        am = jnp.where(upd, jnp.full(shp, c, jnp.int32), am)
        return mx, am

    scores, labels = jax.lax.fori_loop(
        0, C, cls_body,
        (jnp.full(shp, _NEG, jnp.float32), jnp.zeros(shp, jnp.int32)))
    lab_ref[...] = labels

    x1 = bx_ref[0]
    y1 = bx_ref[1]
    x2 = bx_ref[2]
    y2 = bx_ref[3]
    areas = (x2 - x1) * (y2 - y1)

    row_i = jax.lax.broadcasted_iota(jnp.int32, shp, 0)
    lane_i = jax.lax.broadcasted_iota(jnp.int32, shp, 1)
    lin_i = row_i * _LANES + lane_i
    lane1 = jax.lax.broadcasted_iota(jnp.int32, (1, _LANES), 1)

    ob_ref[...] = jnp.full((_MAX_DET, 4), -1.0, jnp.float32)
    os_ref[...] = jnp.full((_MAX_DET, 1), -1.0, jnp.float32)
    ol_ref[...] = jnp.full((_MAX_DET, 1), -1, jnp.int32)

    cur0 = jnp.where(scores > _SCORE_THR, scores, _NEG)

    def body(t, cur):
        m = jnp.max(cur)
        alive = m > _NEG
        lin = jnp.min(jnp.where(cur == m, lin_i, _BIG))
        lin = jnp.where(alive, lin, 0)
        r = lin // _LANES
        c = lin % _LANES
        bxr = bx_ref[:, pl.ds(r, 1), :]
        selc = lane1 == c
        x1i = jnp.sum(jnp.where(selc, bxr[0], 0.0))
        y1i = jnp.sum(jnp.where(selc, bxr[1], 0.0))
        x2i = jnp.sum(jnp.where(selc, bxr[2], 0.0))
        y2i = jnp.sum(jnp.where(selc, bxr[3], 0.0))
        ai = (x2i - x1i) * (y2i - y1i)

        xx1 = jnp.maximum(x1i, x1)
        yy1 = jnp.maximum(y1i, y1)
        xx2 = jnp.minimum(x2i, x2)
        yy2 = jnp.minimum(y2i, y2)
        inter = jnp.maximum(0.0, xx2 - xx1) * jnp.maximum(0.0, yy2 - yy1)
        iou = inter / (ai + areas - inter + 1e-8)
        cur = jnp.where((iou > _IOU_THR) | (lin_i == lin), _NEG, cur)

        @pl.when(alive)
        def _store():
            row = jnp.concatenate(
                [jnp.full((1, 1), v, jnp.float32)
                 for v in (x1i, y1i, x2i, y2i)], axis=1)
            ob_ref[pl.ds(t, 1), :] = row
            os_ref[pl.ds(t, 1), :] = jnp.full((1, 1), m, jnp.float32)
            lrow = lab_ref[pl.ds(r, 1), :]
            li = jnp.sum(jnp.where(selc, lrow, 0))
            ol_ref[pl.ds(t, 1), :] = jnp.full((1, 1), li, jnp.int32)

        return cur

    jax.lax.fori_loop(0, _MAX_DET, body, cur0)


@jax.jit
def kernel(boxes, classification):
    B, N, C = classification.shape
    pad = _NPAD - N
    cls_t = jnp.pad(classification, ((0, 0), (0, pad), (0, 0)),
                    constant_values=-1.0)
    cls_t = cls_t.transpose(0, 2, 1).reshape(B, C, _ROWS, _LANES)
    bx = jnp.pad(boxes, ((0, 0), (0, pad), (0, 0)))
    bx = bx.transpose(0, 2, 1).reshape(B, 4, _ROWS, _LANES)

    ob, os_, ol = pl.pallas_call(
        _nms_kernel,
        grid=(B,),
        in_specs=[
            pl.BlockSpec((None, C, _ROWS, _LANES), lambda b: (b, 0, 0, 0)),
            pl.BlockSpec((None, 4, _ROWS, _LANES), lambda b: (b, 0, 0, 0)),
        ],
        out_specs=[
            pl.BlockSpec((None, _MAX_DET, 4), lambda b: (b, 0, 0)),
            pl.BlockSpec((None, _MAX_DET, 1), lambda b: (b, 0, 0)),
            pl.BlockSpec((None, _MAX_DET, 1), lambda b: (b, 0, 0)),
        ],
        out_shape=[
            jax.ShapeDtypeStruct((B, _MAX_DET, 4), jnp.float32),
            jax.ShapeDtypeStruct((B, _MAX_DET, 1), jnp.float32),
            jax.ShapeDtypeStruct((B, _MAX_DET, 1), jnp.int32),
        ],
        scratch_shapes=[
            pltpu.VMEM((_ROWS, _LANES), jnp.int32),
        ],
    )(cls_t, bx)
    return ob, os_.reshape(B, _MAX_DET), ol.reshape(B, _MAX_DET)


# speedup vs baseline: 1.5905x; 1.1150x over previous
import jax
import jax.numpy as jnp
from jax.experimental import pallas as pl
from jax.experimental.pallas import tpu as pltpu

_MAX_DET = 300
_SCORE_THR = 0.05
_IOU_THR = 0.5
_ROWS = 160
_LANES = 128
_NPAD = _ROWS * _LANES
_NSLOT = 1024
_NEG = float("-inf")
_BIG = 2**30
_UNROLL = 8


def _nms_kernel(cls_ref, bx_ref, kx1_ref, ky1_ref, kx2_ref, ky2_ref,
                ksc_ref, klb_ref, ka_ref):
    C = cls_ref.shape[0]
    shp = (_ROWS, _LANES)

    def cls_body(c, carry):
        mx, am = carry
        x = cls_ref[c]
        upd = x > mx
        mx = jnp.where(upd, x, mx)
        am = jnp.where(upd, jnp.full(shp, c, jnp.int32), am)
        return mx, am

    scores, labels = jax.lax.fori_loop(
        0, C, cls_body,
        (jnp.full(shp, _NEG, jnp.float32), jnp.zeros(shp, jnp.int32)))

    x1 = bx_ref[0]
    y1 = bx_ref[1]
    x2 = bx_ref[2]
    y2 = bx_ref[3]

    row_i = jax.lax.broadcasted_iota(jnp.int32, shp, 0)
    lane_i = jax.lax.broadcasted_iota(jnp.int32, shp, 1)
    lin_i = row_i * _LANES + lane_i
    ksub = jax.lax.broadcasted_iota(jnp.int32, (8, _LANES), 0)
    klane = jax.lax.broadcasted_iota(jnp.int32, (8, _LANES), 1)
    kiota = ksub * _LANES + klane

    kx1_ref[...] = jnp.full((8, _LANES), -1.0, jnp.float32)
    ky1_ref[...] = jnp.full((8, _LANES), -1.0, jnp.float32)
    kx2_ref[...] = jnp.full((8, _LANES), -1.0, jnp.float32)
    ky2_ref[...] = jnp.full((8, _LANES), -1.0, jnp.float32)
    ksc_ref[...] = jnp.full((8, _LANES), -1.0, jnp.float32)
    klb_ref[...] = jnp.full((8, _LANES), -1, jnp.int32)
    ka_ref[...] = jnp.zeros((8, _LANES), jnp.float32)

    cur0 = jnp.where(scores > _SCORE_THR, scores, _NEG)

    def pop(cur, cnt):
        m = jnp.max(cur, axis=(0, 1), keepdims=True)
        alive = m > _NEG
        lin = jnp.min(jnp.where(cur == m, lin_i, _BIG),
                      axis=(0, 1), keepdims=True)
        sel = lin_i == lin
        cur = jnp.where(sel, _NEG, cur)
        fsel = sel.astype(jnp.float32)
        x1i = jnp.sum(fsel * x1, axis=(0, 1), keepdims=True)
        y1i = jnp.sum(fsel * y1, axis=(0, 1), keepdims=True)
        x2i = jnp.sum(fsel * x2, axis=(0, 1), keepdims=True)
        y2i = jnp.sum(fsel * y2, axis=(0, 1), keepdims=True)
        li = jnp.sum(jnp.where(sel, labels, 0), axis=(0, 1), keepdims=True)
        ai = (x2i - x1i) * (y2i - y1i)

        kx1 = kx1_ref[...]
        ky1 = ky1_ref[...]
        kx2 = kx2_ref[...]
        ky2 = ky2_ref[...]
        ka = ka_ref[...]
        xx1 = jnp.maximum(kx1, x1i)
        yy1 = jnp.maximum(ky1, y1i)
        xx2 = jnp.minimum(kx2, x2i)
        yy2 = jnp.minimum(ky2, y2i)
        inter = jnp.maximum(0.0, xx2 - xx1) * jnp.maximum(0.0, yy2 - yy1)
        iou = inter / (ka + ai - inter + 1e-8)
        supp = jnp.any(iou > _IOU_THR, axis=(0, 1), keepdims=True)
        accept = alive & jnp.logical_not(supp)

        put = accept & (kiota == cnt)
        kx1_ref[...] = jnp.where(put, x1i, kx1)
        ky1_ref[...] = jnp.where(put, y1i, ky1)
        kx2_ref[...] = jnp.where(put, x2i, kx2)
        ky2_ref[...] = jnp.where(put, y2i, ky2)
        ka_ref[...] = jnp.where(put, ai, ka)
        ksc_ref[...] = jnp.where(put, m, ksc_ref[...])
        klb_ref[...] = jnp.where(put, li, klb_ref[...])
        cnt = cnt + accept.astype(jnp.int32)
        return cur, cnt, m

    def cond(carry):
        return carry[2]

    def body(carry):
        cur, cnt, _ = carry
        m = None
        for _ in range(_UNROLL):
            cur, cnt, m = pop(cur, cnt)
        s_m = jnp.sum(m)
        s_cnt = jnp.sum(cnt)
        go = (s_m > _NEG) & (s_cnt < _MAX_DET)
        return cur, cnt, go

    jax.lax.while_loop(
        cond, body, (cur0, jnp.zeros((1, 1), jnp.int32), jnp.bool_(True)))


@jax.jit
def kernel(boxes, classification):
    B, N, C = classification.shape
    pad = _NPAD - N
    cls_t = jnp.pad(classification, ((0, 0), (0, pad), (0, 0)),
                    constant_values=-1.0)
    cls_t = cls_t.transpose(0, 2, 1).reshape(B, C, _ROWS, _LANES)
    bx = jnp.pad(boxes, ((0, 0), (0, pad), (0, 0)))
    bx = bx.transpose(0, 2, 1).reshape(B, 4, _ROWS, _LANES)

    kshape = jax.ShapeDtypeStruct((B, 8, _LANES), jnp.float32)
    kspec = pl.BlockSpec((None, 8, _LANES), lambda b: (b, 0, 0))
    kx1, ky1, kx2, ky2, ksc, klb = pl.pallas_call(
        _nms_kernel,
        grid=(B,),
        in_specs=[
            pl.BlockSpec((None, C, _ROWS, _LANES), lambda b: (b, 0, 0, 0)),
            pl.BlockSpec((None, 4, _ROWS, _LANES), lambda b: (b, 0, 0, 0)),
        ],
        out_specs=[kspec] * 6,
        out_shape=[kshape, kshape, kshape, kshape, kshape,
                   jax.ShapeDtypeStruct((B, 8, _LANES), jnp.int32)],
        scratch_shapes=[
            pltpu.VMEM((8, _LANES), jnp.float32),
        ],
    )(cls_t, bx)
    ob = jnp.stack([a.reshape(B, _NSLOT)[:, :_MAX_DET]
                    for a in (kx1, ky1, kx2, ky2)], axis=-1)
    return (ob, ksc.reshape(B, _NSLOT)[:, :_MAX_DET],
            klb.reshape(B, _NSLOT)[:, :_MAX_DET])
